# Initial kernel scaffold; baseline (speedup 1.0000x reference)
import jax
import jax.numpy as jnp
from jax.experimental import pallas as pl


def kernel(tokens, table):
    raise NotImplementedError("write your pallas kernel here")


# speedup vs baseline: 3.9449x; 3.9449x over previous
import functools
import math

import jax
import jax.numpy as jnp
from jax import lax
from jax.experimental import pallas as pl
from jax.experimental.pallas import tpu as pltpu
from jax.experimental.pallas import tpu_sc as plsc

D_MODEL = 64
VOCAB_ROWS = 100001
SCALE = math.sqrt(D_MODEL)

NUM_CORES = 2
NUM_SUBCORES = 16
NUM_WORKERS = NUM_CORES * NUM_SUBCORES

CHUNK = 128
NBUF = 8


_PRESCALE_BLK = 8192


def _prescale_body(x_ref, o_ref):
    row0 = pl.program_id(0) * _PRESCALE_BLK
    rid = lax.broadcasted_iota(jnp.int32, x_ref.shape, 0) + row0
    o_ref[...] = x_ref[...] * jnp.where(rid == 0, 0.0, jnp.float32(SCALE))


def _prescale_table(table):
    nblk = pl.cdiv(table.shape[0], _PRESCALE_BLK)
    return pl.pallas_call(
        _prescale_body,
        grid=(nblk,),
        in_specs=[pl.BlockSpec((_PRESCALE_BLK, D_MODEL), lambda i: (i, 0))],
        out_specs=pl.BlockSpec((_PRESCALE_BLK, D_MODEL), lambda i: (i, 0)),
        out_shape=jax.ShapeDtypeStruct(table.shape, jnp.float32),
    )(table)


def _make_gather(num_tokens):
    assert num_tokens % (NUM_WORKERS * CHUNK) == 0
    per_worker = num_tokens // NUM_WORKERS
    n_chunks = per_worker // CHUNK
    assert n_chunks % NBUF == 0
    n_groups = n_chunks // NBUF

    mesh = plsc.VectorSubcoreMesh(
        core_axis_name="c", subcore_axis_name="s",
        num_cores=NUM_CORES, num_subcores=NUM_SUBCORES)

    @functools.partial(
        pl.kernel,
        out_type=jax.ShapeDtypeStruct((num_tokens, D_MODEL), jnp.float32),
        mesh=mesh,
        compiler_params=pltpu.CompilerParams(use_tc_tiling_on_sc=False),
        scratch_types=[
            pltpu.VMEM((n_chunks, CHUNK), jnp.int32),
            pltpu.VMEM((NBUF, CHUNK, D_MODEL), jnp.float32),
            pltpu.SemaphoreType.DMA,
            pltpu.SemaphoreType.DMA,
        ],
    )
    def gather_kernel(tok_hbm, table_hbm, out_hbm, idx_v, rows_v, gsem, osem):
        wid = lax.axis_index("s") * NUM_CORES + lax.axis_index("c")
        row_base = wid * per_worker
        pltpu.sync_copy(tok_hbm.at[pl.ds(wid * n_chunks, n_chunks)], idx_v)

        def group(g, _):
            j0 = g * NBUF
            gathers = []
            for b in range(NBUF):
                dma = pltpu.make_async_copy(
                    table_hbm.at[idx_v.at[j0 + b]], rows_v.at[b], gsem)
                dma.start()
                gathers.append(dma)
            outs = []
            for b in range(NBUF):
                gathers[b].wait()
                dma = pltpu.make_async_copy(
                    rows_v.at[b],
                    out_hbm.at[pl.ds(row_base + (j0 + b) * CHUNK, CHUNK)],
                    osem)
                dma.start()
                outs.append(dma)
            for b in range(NBUF):
                outs[b].wait()
            return 0

        lax.fori_loop(0, n_groups, group, 0)

    return gather_kernel


def kernel(tokens, table):
    batch, seqlen = tokens.shape
    num_tokens = batch * seqlen
    idx = tokens.reshape(num_tokens // CHUNK, CHUNK).astype(jnp.int32)
    scaled = _prescale_table(table)
    out = _make_gather(num_tokens)(idx, scaled)
    return out.reshape(batch, seqlen, D_MODEL)
